# Initial kernel scaffold; baseline (speedup 1.0000x reference)
import jax
import jax.numpy as jnp
from jax.experimental import pallas as pl


def kernel(x, edge_index, batch, g1_l1_W1, g1_l1_b1, g1_l1_W2, g1_l1_b2, g1_l2_W1, g1_l2_b1, g1_l2_W2, g1_l2_b2, g2_l1_W1, g2_l1_b1, g2_l1_W2, g2_l1_b2, g2_l2_W1, g2_l2_b1, g2_l2_W2, g2_l2_b2, ln_W, ln_b):
    raise NotImplementedError("write your pallas kernel here")


# speedup vs baseline: 10.6473x; 10.6473x over previous
import functools

import jax
import jax.numpy as jnp
from jax import lax
from jax.experimental import pallas as pl
from jax.experimental.pallas import tpu as pltpu
from jax.experimental.pallas import tpu_sc as plsc

N = 10000
E = 320000
D = 128
G = 64
NB = 5
BLK = N // NB
NC = 2
NS = 16
NW = NC * NS
EPT = E // NW
CH = 128
NFULL = EPT // CH
TAIL = EPT - NFULL * CH
ROWS_A = 632
ROWS_B = 624

_HI = jax.lax.Precision.HIGHEST


def _dot(a, b):
    return jnp.dot(a, b, precision=_HI, preferred_element_type=jnp.float32)


def _mm_body(x_ref, w_ref, o_ref):
    o_ref[...] = _dot(x_ref[...], w_ref[...])


def _mid_body(z_ref, p_ref, bd1_ref, bd2_ref, b1_ref, b2_ref, o_ref):
    u = z_ref[...] + p_ref[0] + p_ref[1] + b1_ref[...]
    a = jnp.maximum(u, 0.0)
    h = jnp.maximum(_dot(a, bd1_ref[...]) + b2_ref[...], 0.0)
    o_ref[...] = _dot(h, bd2_ref[...])


def _fin_body(z_ref, p_ref, bd3_ref, b3_ref, b4_ref, lnw_ref, lnb_ref,
              bt_ref, o_ref, acc_ref):
    i = pl.program_id(0)
    v = jnp.maximum(z_ref[...] + p_ref[0] + p_ref[1] + b3_ref[...], 0.0)
    h2 = jnp.maximum(_dot(v, bd3_ref[...]) + b4_ref[...], 0.0)
    bt = bt_ref[0]
    onehot = (lax.broadcasted_iota(jnp.int32, (G, BLK), 0) == bt
              ).astype(jnp.float32)
    contrib = _dot(onehot, h2)

    @pl.when(i == 0)
    def _():
        acc_ref[...] = contrib

    @pl.when(i > 0)
    def _():
        acc_ref[...] += contrib

    @pl.when(i == NB - 1)
    def _():
        o_ref[...] = _dot(acc_ref[...], lnw_ref[...]) + lnb_ref[...]


def _tc_mm(x, w):
    return pl.pallas_call(
        _mm_body,
        grid=(NB,),
        in_specs=[
            pl.BlockSpec((BLK, D), lambda i: (i, 0)),
            pl.BlockSpec((D, D), lambda i: (0, 0)),
        ],
        out_specs=pl.BlockSpec((BLK, D), lambda i: (i, 0)),
        out_shape=jax.ShapeDtypeStruct((N, D), jnp.float32),
    )(x, w)


def _tc_mid(z, p, bd1, bd2, b1, b2):
    return pl.pallas_call(
        _mid_body,
        grid=(NB,),
        in_specs=[
            pl.BlockSpec((BLK, D), lambda i: (i, 0)),
            pl.BlockSpec((2, BLK, D), lambda i: (0, i, 0)),
            pl.BlockSpec((D, D), lambda i: (0, 0)),
            pl.BlockSpec((D, D), lambda i: (0, 0)),
            pl.BlockSpec((1, D), lambda i: (0, 0)),
            pl.BlockSpec((1, D), lambda i: (0, 0)),
        ],
        out_specs=pl.BlockSpec((BLK, D), lambda i: (i, 0)),
        out_shape=jax.ShapeDtypeStruct((N, D), jnp.float32),
    )(z, p, bd1, bd2, b1, b2)


def _tc_fin(z, p, bd3, b3, b4, lnw, lnb, batch3):
    return pl.pallas_call(
        _fin_body,
        grid=(NB,),
        in_specs=[
            pl.BlockSpec((BLK, D), lambda i: (i, 0)),
            pl.BlockSpec((2, BLK, D), lambda i: (0, i, 0)),
            pl.BlockSpec((D, D), lambda i: (0, 0)),
            pl.BlockSpec((1, D), lambda i: (0, 0)),
            pl.BlockSpec((1, D), lambda i: (0, 0)),
            pl.BlockSpec((D, G), lambda i: (0, 0)),
            pl.BlockSpec((1, G), lambda i: (0, 0)),
            pl.BlockSpec((1, 1, BLK), lambda i: (i, 0, 0)),
        ],
        out_specs=pl.BlockSpec((G, G), lambda i: (0, 0)),
        out_shape=jax.ShapeDtypeStruct((G, G), jnp.float32),
        scratch_shapes=[pltpu.VMEM((G, D), jnp.float32)],
    )(z, p, bd3, b3, b4, lnw, lnb, batch3)


def _sc_body(z_hbm, src_hbm, dst_hbm, out_hbm,
             srcv, dstv, rows, srcv_t, dstv_t, rows_t, acc):
    cid = lax.axis_index("c")
    sid = lax.axis_index("s")
    wid = cid * NS + sid

    @pl.loop(0, CH)
    def _(r):
        @pl.loop(0, D // 16)
        def _(c):
            rows[r, pl.ds(c * 16, 16)] = jnp.zeros((16,), jnp.float32)

    base_r = jnp.where(sid < 2, sid * ROWS_A,
                       2 * ROWS_A + (sid - 2) * ROWS_B)

    @pl.loop(0, 4)
    def _(j):
        pltpu.sync_copy(rows, acc.at[pl.ds(base_r + j * CH, CH)])

    @pl.when(sid < 2)
    def _():
        pltpu.sync_copy(rows.at[pl.ds(0, ROWS_A - 4 * CH)],
                        acc.at[pl.ds(base_r + 4 * CH, ROWS_A - 4 * CH)])

    @pl.when(sid >= 2)
    def _():
        pltpu.sync_copy(rows.at[pl.ds(0, ROWS_B - 4 * CH)],
                        acc.at[pl.ds(base_r + 4 * CH, ROWS_B - 4 * CH)])

    plsc.subcore_barrier()

    base_e = wid * EPT

    @pl.loop(0, NFULL)
    def _(i):
        off = base_e + i * CH
        pltpu.sync_copy(src_hbm.at[pl.ds(off, CH)], srcv)
        pltpu.sync_copy(dst_hbm.at[pl.ds(off, CH)], dstv)
        pltpu.sync_copy(z_hbm.at[srcv], rows)
        pltpu.sync_copy(rows, acc.at[dstv], add=True)

    offt = base_e + NFULL * CH
    pltpu.sync_copy(src_hbm.at[pl.ds(offt, TAIL)], srcv_t)
    pltpu.sync_copy(dst_hbm.at[pl.ds(offt, TAIL)], dstv_t)
    pltpu.sync_copy(z_hbm.at[srcv_t], rows_t)
    pltpu.sync_copy(rows_t, acc.at[dstv_t], add=True)
    plsc.subcore_barrier()

    @pl.when(sid < 2)
    def _():
        pltpu.sync_copy(acc.at[pl.ds(base_r, ROWS_A)],
                        out_hbm.at[cid, pl.ds(base_r, ROWS_A)])

    @pl.when(sid >= 2)
    def _():
        pltpu.sync_copy(acc.at[pl.ds(base_r, ROWS_B)],
                        out_hbm.at[cid, pl.ds(base_r, ROWS_B)])


@functools.partial(
    pl.kernel,
    out_type=jax.ShapeDtypeStruct((NC, N, D), jnp.float32),
    mesh=plsc.VectorSubcoreMesh(core_axis_name="c", subcore_axis_name="s"),
    scratch_types=[
        pltpu.VMEM((CH,), jnp.int32),
        pltpu.VMEM((CH,), jnp.int32),
        pltpu.VMEM((CH, D), jnp.float32),
        pltpu.VMEM((TAIL,), jnp.int32),
        pltpu.VMEM((TAIL,), jnp.int32),
        pltpu.VMEM((TAIL, D), jnp.float32),
        pltpu.VMEM_SHARED((N, D), jnp.float32),
    ],
)
def _sc_edge_segsum(z_hbm, src_hbm, dst_hbm, out_hbm,
                    srcv, dstv, rows, srcv_t, dstv_t, rows_t, acc):
    _sc_body(z_hbm, src_hbm, dst_hbm, out_hbm,
             srcv, dstv, rows, srcv_t, dstv_t, rows_t, acc)


def kernel(x, edge_index, batch,
           g1_l1_W1, g1_l1_b1, g1_l1_W2, g1_l1_b2,
           g1_l2_W1, g1_l2_b1, g1_l2_W2, g1_l2_b2,
           g2_l1_W1, g2_l1_b1, g2_l1_W2, g2_l1_b2,
           g2_l2_W1, g2_l2_b1, g2_l2_W2, g2_l2_b2,
           ln_W, ln_b):
    f32 = jnp.float32
    H = 64

    src = edge_index[0]
    dst = edge_index[1]

    z64 = jnp.zeros((H, H), f32)
    z1_64 = jnp.zeros((D - 1, H), f32)
    z64_1 = jnp.zeros((1, H), f32)
    wcat = jnp.concatenate([
        jnp.concatenate([z1_64, g2_l1_W1], axis=1),
        jnp.concatenate([g1_l1_W1, z64_1], axis=1),
    ], axis=0)
    bd1 = jnp.concatenate([
        jnp.concatenate([g1_l1_W2, z64], axis=1),
        jnp.concatenate([z64, g2_l1_W2], axis=1),
    ], axis=0)
    bd2 = jnp.concatenate([
        jnp.concatenate([g1_l2_W1, z64], axis=1),
        jnp.concatenate([z64, g2_l2_W1], axis=1),
    ], axis=0)
    bd3 = jnp.concatenate([
        jnp.concatenate([g1_l2_W2, z64], axis=1),
        jnp.concatenate([z64, g2_l2_W2], axis=1),
    ], axis=0)
    b1 = jnp.concatenate([g1_l1_b1, g2_l1_b1]).reshape(1, D)
    b2 = jnp.concatenate([g1_l1_b2, g2_l1_b2]).reshape(1, D)
    b3 = jnp.concatenate([g1_l2_b1, g2_l2_b1]).reshape(1, D)
    b4 = jnp.concatenate([g1_l2_b2, g2_l2_b2]).reshape(1, D)
    lnb = ln_b.reshape(1, G)
    batch3 = batch.reshape(NB, 1, BLK)

    z1 = _tc_mm(x, wcat)
    p1 = _sc_edge_segsum(z1, src, dst)
    z2 = _tc_mid(z1, p1, bd1, bd2, b1, b2)
    p2 = _sc_edge_segsum(z2, src, dst)
    out = _tc_fin(z2, p2, bd3, b3, b4, ln_W, lnb, batch3)
    return out


# speedup vs baseline: 18.9539x; 1.7802x over previous
import functools

import jax
import jax.numpy as jnp
from jax import lax
from jax.experimental import pallas as pl
from jax.experimental.pallas import tpu as pltpu
from jax.experimental.pallas import tpu_sc as plsc

N = 10000
E = 320000
D = 128
G = 64
NB = 5
BLK = N // NB
NC = 2
NS = 16
NW = NC * NS
EPT = E // NW
CH = 128
NFULL = EPT // CH
TAIL = EPT - NFULL * CH
ROWS_A = 632
ROWS_B = 624

_HI = jax.lax.Precision.HIGHEST


def _dot(a, b):
    return jnp.dot(a, b, precision=_HI, preferred_element_type=jnp.float32)


def _mm_body(x_ref, w_ref, o_ref):
    o_ref[...] = _dot(x_ref[...], w_ref[...])


def _mid_body(z_ref, p_ref, bd1_ref, bd2_ref, b1_ref, b2_ref, o_ref):
    u = z_ref[...] + p_ref[0] + p_ref[1] + b1_ref[...]
    a = jnp.maximum(u, 0.0)
    h = jnp.maximum(_dot(a, bd1_ref[...]) + b2_ref[...], 0.0)
    o_ref[...] = _dot(h, bd2_ref[...])


def _fin_body(z_ref, p_ref, bd3_ref, b3_ref, b4_ref, lnw_ref, lnb_ref,
              bt_ref, o_ref, acc_ref):
    i = pl.program_id(0)
    v = jnp.maximum(z_ref[...] + p_ref[0] + p_ref[1] + b3_ref[...], 0.0)
    h2 = jnp.maximum(_dot(v, bd3_ref[...]) + b4_ref[...], 0.0)
    bt = bt_ref[0]
    onehot = (lax.broadcasted_iota(jnp.int32, (G, BLK), 0) == bt
              ).astype(jnp.float32)
    contrib = _dot(onehot, h2)

    @pl.when(i == 0)
    def _():
        acc_ref[...] = contrib

    @pl.when(i > 0)
    def _():
        acc_ref[...] += contrib

    @pl.when(i == NB - 1)
    def _():
        o_ref[...] = _dot(acc_ref[...], lnw_ref[...]) + lnb_ref[...]


def _tc_mm(x, w):
    return pl.pallas_call(
        _mm_body,
        grid=(NB,),
        in_specs=[
            pl.BlockSpec((BLK, D), lambda i: (i, 0)),
            pl.BlockSpec((D, D), lambda i: (0, 0)),
        ],
        out_specs=pl.BlockSpec((BLK, D), lambda i: (i, 0)),
        out_shape=jax.ShapeDtypeStruct((N, D), jnp.float32),
    )(x, w)


def _tc_mid(z, p, bd1, bd2, b1, b2):
    return pl.pallas_call(
        _mid_body,
        grid=(NB,),
        in_specs=[
            pl.BlockSpec((BLK, D), lambda i: (i, 0)),
            pl.BlockSpec((2, BLK, D), lambda i: (0, i, 0)),
            pl.BlockSpec((D, D), lambda i: (0, 0)),
            pl.BlockSpec((D, D), lambda i: (0, 0)),
            pl.BlockSpec((1, D), lambda i: (0, 0)),
            pl.BlockSpec((1, D), lambda i: (0, 0)),
        ],
        out_specs=pl.BlockSpec((BLK, D), lambda i: (i, 0)),
        out_shape=jax.ShapeDtypeStruct((N, D), jnp.float32),
    )(z, p, bd1, bd2, b1, b2)


def _tc_fin(z, p, bd3, b3, b4, lnw, lnb, batch3):
    return pl.pallas_call(
        _fin_body,
        grid=(NB,),
        in_specs=[
            pl.BlockSpec((BLK, D), lambda i: (i, 0)),
            pl.BlockSpec((2, BLK, D), lambda i: (0, i, 0)),
            pl.BlockSpec((D, D), lambda i: (0, 0)),
            pl.BlockSpec((1, D), lambda i: (0, 0)),
            pl.BlockSpec((1, D), lambda i: (0, 0)),
            pl.BlockSpec((D, G), lambda i: (0, 0)),
            pl.BlockSpec((1, G), lambda i: (0, 0)),
            pl.BlockSpec((1, 1, BLK), lambda i: (i, 0, 0)),
        ],
        out_specs=pl.BlockSpec((G, G), lambda i: (0, 0)),
        out_shape=jax.ShapeDtypeStruct((G, G), jnp.float32),
        scratch_shapes=[pltpu.VMEM((G, D), jnp.float32)],
    )(z, p, bd3, b3, b4, lnw, lnb, batch3)


def _sc_body(z_hbm, src_hbm, dst_hbm, out_hbm,
             srcb, dstb, rows, srcv_t, dstv_t, rows_t,
             sem_g0, sem_g1, sem_i0, sem_i1, acc):
    cid = lax.axis_index("c")
    sid = lax.axis_index("s")
    wid = cid * NS + sid

    @pl.loop(0, CH)
    def _(r):
        @pl.loop(0, D // 16)
        def _(c):
            rows[0, r, pl.ds(c * 16, 16)] = jnp.zeros((16,), jnp.float32)

    base_r = jnp.where(sid < 2, sid * ROWS_A,
                       2 * ROWS_A + (sid - 2) * ROWS_B)

    @pl.loop(0, 4)
    def _(j):
        pltpu.sync_copy(rows.at[0], acc.at[pl.ds(base_r + j * CH, CH)])

    @pl.when(sid < 2)
    def _():
        pltpu.sync_copy(rows.at[0].at[pl.ds(0, ROWS_A - 4 * CH)],
                        acc.at[pl.ds(base_r + 4 * CH, ROWS_A - 4 * CH)])

    @pl.when(sid >= 2)
    def _():
        pltpu.sync_copy(rows.at[0].at[pl.ds(0, ROWS_B - 4 * CH)],
                        acc.at[pl.ds(base_r + 4 * CH, ROWS_B - 4 * CH)])

    plsc.subcore_barrier()

    base_e = wid * EPT
    sem_g = (sem_g0, sem_g1)
    sem_i = (sem_i0, sem_i1)

    def _idx_load(j, b):
        off = base_e + j * CH
        pltpu.async_copy(src_hbm.at[pl.ds(off, CH)], srcb.at[b], sem_i[b])
        pltpu.async_copy(dst_hbm.at[pl.ds(off, CH)], dstb.at[b], sem_i[b])

    def _idx_wait(j, b):
        off = base_e + j * CH
        pltpu.make_async_copy(
            src_hbm.at[pl.ds(off, CH)], srcb.at[b], sem_i[b]).wait()
        pltpu.make_async_copy(
            dst_hbm.at[pl.ds(off, CH)], dstb.at[b], sem_i[b]).wait()

    def _gather(j, b):
        return pltpu.make_async_copy(
            z_hbm.at[srcb.at[b]], rows.at[b], sem_g[b])

    _idx_load(0, 0)
    _idx_wait(0, 0)
    _gather(0, 0).start()
    _idx_load(1, 1)

    def _slot(j, b):
        @pl.when(j + 1 < NFULL)
        def _():
            _idx_wait(j + 1, 1 - b)
            _gather(j + 1, 1 - b).start()
        _gather(j, b).wait()
        pltpu.sync_copy(rows.at[b], acc.at[dstb.at[b]], add=True)

        @pl.when(j + 2 < NFULL)
        def _():
            _idx_load(j + 2, b)

    @pl.loop(0, NFULL // 2)
    def _(g):
        _slot(2 * g, 0)
        _slot(2 * g + 1, 1)

    offt = base_e + NFULL * CH
    pltpu.sync_copy(src_hbm.at[pl.ds(offt, TAIL)], srcv_t)
    pltpu.sync_copy(dst_hbm.at[pl.ds(offt, TAIL)], dstv_t)
    pltpu.sync_copy(z_hbm.at[srcv_t], rows_t)
    pltpu.sync_copy(rows_t, acc.at[dstv_t], add=True)
    plsc.subcore_barrier()

    @pl.when(sid < 2)
    def _():
        pltpu.sync_copy(acc.at[pl.ds(base_r, ROWS_A)],
                        out_hbm.at[cid, pl.ds(base_r, ROWS_A)])

    @pl.when(sid >= 2)
    def _():
        pltpu.sync_copy(acc.at[pl.ds(base_r, ROWS_B)],
                        out_hbm.at[cid, pl.ds(base_r, ROWS_B)])


@functools.partial(
    pl.kernel,
    out_type=jax.ShapeDtypeStruct((NC, N, D), jnp.float32),
    mesh=plsc.VectorSubcoreMesh(core_axis_name="c", subcore_axis_name="s"),
    scratch_types=[
        pltpu.VMEM((2, CH), jnp.int32),
        pltpu.VMEM((2, CH), jnp.int32),
        pltpu.VMEM((2, CH, D), jnp.float32),
        pltpu.VMEM((TAIL,), jnp.int32),
        pltpu.VMEM((TAIL,), jnp.int32),
        pltpu.VMEM((TAIL, D), jnp.float32),
        pltpu.SemaphoreType.DMA,
        pltpu.SemaphoreType.DMA,
        pltpu.SemaphoreType.DMA,
        pltpu.SemaphoreType.DMA,
        pltpu.VMEM_SHARED((N, D), jnp.float32),
    ],
)
def _sc_edge_segsum(z_hbm, src_hbm, dst_hbm, out_hbm,
                    srcb, dstb, rows, srcv_t, dstv_t, rows_t,
                    sem_g0, sem_g1, sem_i0, sem_i1, acc):
    _sc_body(z_hbm, src_hbm, dst_hbm, out_hbm,
             srcb, dstb, rows, srcv_t, dstv_t, rows_t,
             sem_g0, sem_g1, sem_i0, sem_i1, acc)


def kernel(x, edge_index, batch,
           g1_l1_W1, g1_l1_b1, g1_l1_W2, g1_l1_b2,
           g1_l2_W1, g1_l2_b1, g1_l2_W2, g1_l2_b2,
           g2_l1_W1, g2_l1_b1, g2_l1_W2, g2_l1_b2,
           g2_l2_W1, g2_l2_b1, g2_l2_W2, g2_l2_b2,
           ln_W, ln_b):
    f32 = jnp.float32
    H = 64

    src = edge_index[0]
    dst = edge_index[1]

    z64 = jnp.zeros((H, H), f32)
    z1_64 = jnp.zeros((D - 1, H), f32)
    z64_1 = jnp.zeros((1, H), f32)
    wcat = jnp.concatenate([
        jnp.concatenate([z1_64, g2_l1_W1], axis=1),
        jnp.concatenate([g1_l1_W1, z64_1], axis=1),
    ], axis=0)
    bd1 = jnp.concatenate([
        jnp.concatenate([g1_l1_W2, z64], axis=1),
        jnp.concatenate([z64, g2_l1_W2], axis=1),
    ], axis=0)
    bd2 = jnp.concatenate([
        jnp.concatenate([g1_l2_W1, z64], axis=1),
        jnp.concatenate([z64, g2_l2_W1], axis=1),
    ], axis=0)
    bd3 = jnp.concatenate([
        jnp.concatenate([g1_l2_W2, z64], axis=1),
        jnp.concatenate([z64, g2_l2_W2], axis=1),
    ], axis=0)
    b1 = jnp.concatenate([g1_l1_b1, g2_l1_b1]).reshape(1, D)
    b2 = jnp.concatenate([g1_l1_b2, g2_l1_b2]).reshape(1, D)
    b3 = jnp.concatenate([g1_l2_b1, g2_l2_b1]).reshape(1, D)
    b4 = jnp.concatenate([g1_l2_b2, g2_l2_b2]).reshape(1, D)
    lnb = ln_b.reshape(1, G)
    batch3 = batch.reshape(NB, 1, BLK)

    z1 = _tc_mm(x, wcat)
    p1 = _sc_edge_segsum(z1, src, dst)
    z2 = _tc_mid(z1, p1, bd1, bd2, b1, b2)
    p2 = _sc_edge_segsum(z2, src, dst)
    out = _tc_fin(z2, p2, bd3, b3, b4, ln_W, lnb, batch3)
    return out


# speedup vs baseline: 19.2405x; 1.0151x over previous
import functools

import jax
import jax.numpy as jnp
from jax import lax
from jax.experimental import pallas as pl
from jax.experimental.pallas import tpu as pltpu
from jax.experimental.pallas import tpu_sc as plsc

N = 10000
E = 320000
D = 128
G = 64
NB = 5
BLK = N // NB
NC = 2
NS = 16
NW = NC * NS
EPT = E // NW
CH = 128
NFULL = EPT // CH
TAIL = EPT - NFULL * CH
ROWS_A = 632
ROWS_B = 624

_HI = jax.lax.Precision.HIGHEST


def _dot(a, b):
    return jnp.dot(a, b, precision=_HI, preferred_element_type=jnp.float32)


def _mid_body(x_ref, p_ref, wcat_ref, bd1_ref, bd2_ref, b1_ref, b2_ref,
              o_ref):
    u = x_ref[...] + p_ref[0] + p_ref[1]
    z1 = _dot(u, wcat_ref[...])
    a = jnp.maximum(z1 + b1_ref[...], 0.0)
    h = jnp.maximum(_dot(a, bd1_ref[...]) + b2_ref[...], 0.0)
    o_ref[...] = _dot(h, bd2_ref[...])


def _fin_body(z_ref, p_ref, bd3_ref, b3_ref, b4_ref, lnw_ref, lnb_ref,
              bt_ref, o_ref, acc_ref):
    i = pl.program_id(0)
    v = jnp.maximum(z_ref[...] + p_ref[0] + p_ref[1] + b3_ref[...], 0.0)
    h2 = jnp.maximum(_dot(v, bd3_ref[...]) + b4_ref[...], 0.0)
    bt = bt_ref[0]
    onehot = (lax.broadcasted_iota(jnp.int32, (G, BLK), 0) == bt
              ).astype(jnp.float32)
    contrib = _dot(onehot, h2)

    @pl.when(i == 0)
    def _():
        acc_ref[...] = contrib

    @pl.when(i > 0)
    def _():
        acc_ref[...] += contrib

    @pl.when(i == NB - 1)
    def _():
        o_ref[...] = _dot(acc_ref[...], lnw_ref[...]) + lnb_ref[...]


def _tc_mid(x, p, wcat, bd1, bd2, b1, b2):
    return pl.pallas_call(
        _mid_body,
        grid=(NB,),
        in_specs=[
            pl.BlockSpec((BLK, D), lambda i: (i, 0)),
            pl.BlockSpec((2, BLK, D), lambda i: (0, i, 0)),
            pl.BlockSpec((D, D), lambda i: (0, 0)),
            pl.BlockSpec((D, D), lambda i: (0, 0)),
            pl.BlockSpec((D, D), lambda i: (0, 0)),
            pl.BlockSpec((1, D), lambda i: (0, 0)),
            pl.BlockSpec((1, D), lambda i: (0, 0)),
        ],
        out_specs=pl.BlockSpec((BLK, D), lambda i: (i, 0)),
        out_shape=jax.ShapeDtypeStruct((N, D), jnp.float32),
    )(x, p, wcat, bd1, bd2, b1, b2)


def _tc_fin(z, p, bd3, b3, b4, lnw, lnb, batch3):
    return pl.pallas_call(
        _fin_body,
        grid=(NB,),
        in_specs=[
            pl.BlockSpec((BLK, D), lambda i: (i, 0)),
            pl.BlockSpec((2, BLK, D), lambda i: (0, i, 0)),
            pl.BlockSpec((D, D), lambda i: (0, 0)),
            pl.BlockSpec((1, D), lambda i: (0, 0)),
            pl.BlockSpec((1, D), lambda i: (0, 0)),
            pl.BlockSpec((D, G), lambda i: (0, 0)),
            pl.BlockSpec((1, G), lambda i: (0, 0)),
            pl.BlockSpec((1, 1, BLK), lambda i: (i, 0, 0)),
        ],
        out_specs=pl.BlockSpec((G, G), lambda i: (0, 0)),
        out_shape=jax.ShapeDtypeStruct((G, G), jnp.float32),
        scratch_shapes=[pltpu.VMEM((G, D), jnp.float32)],
    )(z, p, bd3, b3, b4, lnw, lnb, batch3)


def _sc_body(z_hbm, src_hbm, dst_hbm, out_hbm,
             srcb, dstb, rows, srcv_t, dstv_t, rows_t,
             sem_g0, sem_g1, sem_i0, sem_i1, acc):
    cid = lax.axis_index("c")
    sid = lax.axis_index("s")
    wid = cid * NS + sid

    @pl.loop(0, CH)
    def _(r):
        @pl.loop(0, D // 16)
        def _(c):
            rows[0, r, pl.ds(c * 16, 16)] = jnp.zeros((16,), jnp.float32)

    base_r = jnp.where(sid < 2, sid * ROWS_A,
                       2 * ROWS_A + (sid - 2) * ROWS_B)

    @pl.loop(0, 4)
    def _(j):
        pltpu.sync_copy(rows.at[0], acc.at[pl.ds(base_r + j * CH, CH)])

    @pl.when(sid < 2)
    def _():
        pltpu.sync_copy(rows.at[0].at[pl.ds(0, ROWS_A - 4 * CH)],
                        acc.at[pl.ds(base_r + 4 * CH, ROWS_A - 4 * CH)])

    @pl.when(sid >= 2)
    def _():
        pltpu.sync_copy(rows.at[0].at[pl.ds(0, ROWS_B - 4 * CH)],
                        acc.at[pl.ds(base_r + 4 * CH, ROWS_B - 4 * CH)])

    plsc.subcore_barrier()

    base_e = wid * EPT
    sem_g = (sem_g0, sem_g1)
    sem_i = (sem_i0, sem_i1)

    def _idx_load(j, b):
        off = base_e + j * CH
        pltpu.async_copy(src_hbm.at[pl.ds(off, CH)], srcb.at[b], sem_i[b])
        pltpu.async_copy(dst_hbm.at[pl.ds(off, CH)], dstb.at[b], sem_i[b])

    def _idx_wait(j, b):
        off = base_e + j * CH
        pltpu.make_async_copy(
            src_hbm.at[pl.ds(off, CH)], srcb.at[b], sem_i[b]).wait()
        pltpu.make_async_copy(
            dst_hbm.at[pl.ds(off, CH)], dstb.at[b], sem_i[b]).wait()

    def _gather(j, b):
        return pltpu.make_async_copy(
            z_hbm.at[srcb.at[b]], rows.at[b], sem_g[b])

    _idx_load(0, 0)
    _idx_wait(0, 0)
    _gather(0, 0).start()
    _idx_load(1, 1)

    def _slot(j, b):
        @pl.when(j + 1 < NFULL)
        def _():
            _idx_wait(j + 1, 1 - b)
            _gather(j + 1, 1 - b).start()
        _gather(j, b).wait()
        pltpu.sync_copy(rows.at[b], acc.at[dstb.at[b]], add=True)

        @pl.when(j + 2 < NFULL)
        def _():
            _idx_load(j + 2, b)

    @pl.loop(0, NFULL // 2)
    def _(g):
        _slot(2 * g, 0)
        _slot(2 * g + 1, 1)

    offt = base_e + NFULL * CH
    pltpu.sync_copy(src_hbm.at[pl.ds(offt, TAIL)], srcv_t)
    pltpu.sync_copy(dst_hbm.at[pl.ds(offt, TAIL)], dstv_t)
    pltpu.sync_copy(z_hbm.at[srcv_t], rows_t)
    pltpu.sync_copy(rows_t, acc.at[dstv_t], add=True)
    plsc.subcore_barrier()

    @pl.when(sid < 2)
    def _():
        pltpu.sync_copy(acc.at[pl.ds(base_r, ROWS_A)],
                        out_hbm.at[cid, pl.ds(base_r, ROWS_A)])

    @pl.when(sid >= 2)
    def _():
        pltpu.sync_copy(acc.at[pl.ds(base_r, ROWS_B)],
                        out_hbm.at[cid, pl.ds(base_r, ROWS_B)])


@functools.partial(
    pl.kernel,
    out_type=jax.ShapeDtypeStruct((NC, N, D), jnp.float32),
    mesh=plsc.VectorSubcoreMesh(core_axis_name="c", subcore_axis_name="s"),
    scratch_types=[
        pltpu.VMEM((2, CH), jnp.int32),
        pltpu.VMEM((2, CH), jnp.int32),
        pltpu.VMEM((2, CH, D), jnp.float32),
        pltpu.VMEM((TAIL,), jnp.int32),
        pltpu.VMEM((TAIL,), jnp.int32),
        pltpu.VMEM((TAIL, D), jnp.float32),
        pltpu.SemaphoreType.DMA,
        pltpu.SemaphoreType.DMA,
        pltpu.SemaphoreType.DMA,
        pltpu.SemaphoreType.DMA,
        pltpu.VMEM_SHARED((N, D), jnp.float32),
    ],
)
def _sc_edge_segsum(z_hbm, src_hbm, dst_hbm, out_hbm,
                    srcb, dstb, rows, srcv_t, dstv_t, rows_t,
                    sem_g0, sem_g1, sem_i0, sem_i1, acc):
    _sc_body(z_hbm, src_hbm, dst_hbm, out_hbm,
             srcb, dstb, rows, srcv_t, dstv_t, rows_t,
             sem_g0, sem_g1, sem_i0, sem_i1, acc)


def kernel(x, edge_index, batch,
           g1_l1_W1, g1_l1_b1, g1_l1_W2, g1_l1_b2,
           g1_l2_W1, g1_l2_b1, g1_l2_W2, g1_l2_b2,
           g2_l1_W1, g2_l1_b1, g2_l1_W2, g2_l1_b2,
           g2_l2_W1, g2_l2_b1, g2_l2_W2, g2_l2_b2,
           ln_W, ln_b):
    f32 = jnp.float32
    H = 64

    src = edge_index[0]
    dst = edge_index[1]

    z64 = jnp.zeros((H, H), f32)
    z1_64 = jnp.zeros((D - 1, H), f32)
    z64_1 = jnp.zeros((1, H), f32)
    wcat = jnp.concatenate([
        jnp.concatenate([z1_64, g2_l1_W1], axis=1),
        jnp.concatenate([g1_l1_W1, z64_1], axis=1),
    ], axis=0)
    bd1 = jnp.concatenate([
        jnp.concatenate([g1_l1_W2, z64], axis=1),
        jnp.concatenate([z64, g2_l1_W2], axis=1),
    ], axis=0)
    bd2 = jnp.concatenate([
        jnp.concatenate([g1_l2_W1, z64], axis=1),
        jnp.concatenate([z64, g2_l2_W1], axis=1),
    ], axis=0)
    bd3 = jnp.concatenate([
        jnp.concatenate([g1_l2_W2, z64], axis=1),
        jnp.concatenate([z64, g2_l2_W2], axis=1),
    ], axis=0)
    b1 = jnp.concatenate([g1_l1_b1, g2_l1_b1]).reshape(1, D)
    b2 = jnp.concatenate([g1_l1_b2, g2_l1_b2]).reshape(1, D)
    b3 = jnp.concatenate([g1_l2_b1, g2_l2_b1]).reshape(1, D)
    b4 = jnp.concatenate([g1_l2_b2, g2_l2_b2]).reshape(1, D)
    lnb = ln_b.reshape(1, G)
    batch3 = batch.reshape(NB, 1, BLK)

    p1 = _sc_edge_segsum(x, src, dst)
    z2 = _tc_mid(x, p1, wcat, bd1, bd2, b1, b2)
    p2 = _sc_edge_segsum(z2, src, dst)
    out = _tc_fin(z2, p2, bd3, b3, b4, ln_W, lnb, batch3)
    return out


# speedup vs baseline: 21.2246x; 1.1031x over previous
import functools

import jax
import jax.numpy as jnp
from jax import lax
from jax.experimental import pallas as pl
from jax.experimental.pallas import tpu as pltpu
from jax.experimental.pallas import tpu_sc as plsc

N = 10000
E = 320000
D = 128
G = 64
NB = 5
BLK = N // NB
NC = 2
NS = 16
NW = NC * NS
EPT = E // NW
CH = 64
NR = 4
NFULL = EPT // CH
TAIL = EPT - NFULL * CH
ROWS_A = 632
ROWS_B = 624

_HI = jax.lax.Precision.HIGHEST


def _dot(a, b):
    return jnp.dot(a, b, precision=_HI, preferred_element_type=jnp.float32)


def _mid_body(x_ref, p_ref, wcat_ref, bd1_ref, bd2_ref, b1_ref, b2_ref,
              o_ref):
    u = x_ref[...] + p_ref[0] + p_ref[1]
    z1 = _dot(u, wcat_ref[...])
    a = jnp.maximum(z1 + b1_ref[...], 0.0)
    h = jnp.maximum(_dot(a, bd1_ref[...]) + b2_ref[...], 0.0)
    o_ref[...] = _dot(h, bd2_ref[...])


def _fin_body(z_ref, p_ref, bd3_ref, b3_ref, b4_ref, lnw_ref, lnb_ref,
              bt_ref, o_ref, acc_ref):
    i = pl.program_id(0)
    v = jnp.maximum(z_ref[...] + p_ref[0] + p_ref[1] + b3_ref[...], 0.0)
    h2 = jnp.maximum(_dot(v, bd3_ref[...]) + b4_ref[...], 0.0)
    bt = bt_ref[0]
    onehot = (lax.broadcasted_iota(jnp.int32, (G, BLK), 0) == bt
              ).astype(jnp.float32)
    contrib = _dot(onehot, h2)

    @pl.when(i == 0)
    def _():
        acc_ref[...] = contrib

    @pl.when(i > 0)
    def _():
        acc_ref[...] += contrib

    @pl.when(i == NB - 1)
    def _():
        o_ref[...] = _dot(acc_ref[...], lnw_ref[...]) + lnb_ref[...]


def _tc_mid(x, p, wcat, bd1, bd2, b1, b2):
    return pl.pallas_call(
        _mid_body,
        grid=(NB,),
        in_specs=[
            pl.BlockSpec((BLK, D), lambda i: (i, 0)),
            pl.BlockSpec((2, BLK, D), lambda i: (0, i, 0)),
            pl.BlockSpec((D, D), lambda i: (0, 0)),
            pl.BlockSpec((D, D), lambda i: (0, 0)),
            pl.BlockSpec((D, D), lambda i: (0, 0)),
            pl.BlockSpec((1, D), lambda i: (0, 0)),
            pl.BlockSpec((1, D), lambda i: (0, 0)),
        ],
        out_specs=pl.BlockSpec((BLK, D), lambda i: (i, 0)),
        out_shape=jax.ShapeDtypeStruct((N, D), jnp.float32),
    )(x, p, wcat, bd1, bd2, b1, b2)


def _tc_fin(z, p, bd3, b3, b4, lnw, lnb, batch3):
    return pl.pallas_call(
        _fin_body,
        grid=(NB,),
        in_specs=[
            pl.BlockSpec((BLK, D), lambda i: (i, 0)),
            pl.BlockSpec((2, BLK, D), lambda i: (0, i, 0)),
            pl.BlockSpec((D, D), lambda i: (0, 0)),
            pl.BlockSpec((1, D), lambda i: (0, 0)),
            pl.BlockSpec((1, D), lambda i: (0, 0)),
            pl.BlockSpec((D, G), lambda i: (0, 0)),
            pl.BlockSpec((1, G), lambda i: (0, 0)),
            pl.BlockSpec((1, 1, BLK), lambda i: (i, 0, 0)),
        ],
        out_specs=pl.BlockSpec((G, G), lambda i: (0, 0)),
        out_shape=jax.ShapeDtypeStruct((G, G), jnp.float32),
        scratch_shapes=[pltpu.VMEM((G, D), jnp.float32)],
    )(z, p, bd3, b3, b4, lnw, lnb, batch3)


def _sc_body(z_hbm, src_hbm, dst_hbm, out_hbm,
             srcb, dstb, rows, srcv_t, dstv_t, rows_t,
             sem_g, sem_i, sem_s, acc):
    cid = lax.axis_index("c")
    sid = lax.axis_index("s")
    wid = cid * NS + sid

    @pl.loop(0, CH)
    def _(r):
        @pl.loop(0, D // 16)
        def _(c):
            rows[0, r, pl.ds(c * 16, 16)] = jnp.zeros((16,), jnp.float32)

    base_r = jnp.where(sid < 2, sid * ROWS_A,
                       2 * ROWS_A + (sid - 2) * ROWS_B)

    @pl.loop(0, 9)
    def _(j):
        pltpu.sync_copy(rows.at[0], acc.at[pl.ds(base_r + j * CH, CH)])

    @pl.when(sid < 2)
    def _():
        pltpu.sync_copy(rows.at[0].at[pl.ds(0, ROWS_A - 9 * CH)],
                        acc.at[pl.ds(base_r + 9 * CH, ROWS_A - 9 * CH)])

    @pl.when(sid >= 2)
    def _():
        pltpu.sync_copy(rows.at[0].at[pl.ds(0, ROWS_B - 9 * CH)],
                        acc.at[pl.ds(base_r + 9 * CH, ROWS_B - 9 * CH)])

    plsc.subcore_barrier()

    base_e = wid * EPT

    def _idx_load(j, b):
        off = base_e + j * CH
        pltpu.async_copy(src_hbm.at[pl.ds(off, CH)], srcb.at[b], sem_i[b])
        pltpu.async_copy(dst_hbm.at[pl.ds(off, CH)], dstb.at[b], sem_i[b])

    def _idx_wait(j, b):
        off = base_e + j * CH
        pltpu.make_async_copy(
            src_hbm.at[pl.ds(off, CH)], srcb.at[b], sem_i[b]).wait()
        pltpu.make_async_copy(
            dst_hbm.at[pl.ds(off, CH)], dstb.at[b], sem_i[b]).wait()

    def _gather(b):
        return pltpu.make_async_copy(
            z_hbm.at[srcb.at[b]], rows.at[b], sem_g[b])

    def _scatter_start(b):
        pltpu.async_copy(rows.at[b], acc.at[dstb.at[b]], sem_s[b], add=True)

    def _scatter_wait(b):
        pltpu.make_async_copy(
            rows.at[b], acc.at[dstb.at[b]], sem_s[b]).wait()

    _idx_load(0, 0)
    _idx_load(1, 1)
    _idx_wait(0, 0)
    _gather(0).start()

    def _step(c, u):
        @pl.when(c >= 2)
        def _():
            _scatter_wait((u + 2) % NR)

        @pl.when(c + 2 < NFULL)
        def _():
            _idx_load(c + 2, (u + 2) % NR)

        @pl.when(c + 1 < NFULL)
        def _():
            _idx_wait(c + 1, (u + 1) % NR)
            _gather((u + 1) % NR).start()

        _gather(u).wait()
        _scatter_start(u)

    @pl.loop(0, NFULL // NR)
    def _(g):
        for u in range(NR):
            _step(NR * g + u, u)

    _scatter_wait((NFULL - 2) % NR)
    _scatter_wait((NFULL - 1) % NR)

    offt = base_e + NFULL * CH
    pltpu.sync_copy(src_hbm.at[pl.ds(offt, TAIL)], srcv_t)
    pltpu.sync_copy(dst_hbm.at[pl.ds(offt, TAIL)], dstv_t)
    pltpu.sync_copy(z_hbm.at[srcv_t], rows_t)
    pltpu.sync_copy(rows_t, acc.at[dstv_t], add=True)
    plsc.subcore_barrier()

    @pl.when(sid < 2)
    def _():
        pltpu.sync_copy(acc.at[pl.ds(base_r, ROWS_A)],
                        out_hbm.at[cid, pl.ds(base_r, ROWS_A)])

    @pl.when(sid >= 2)
    def _():
        pltpu.sync_copy(acc.at[pl.ds(base_r, ROWS_B)],
                        out_hbm.at[cid, pl.ds(base_r, ROWS_B)])


@functools.partial(
    pl.kernel,
    out_type=jax.ShapeDtypeStruct((NC, N, D), jnp.float32),
    mesh=plsc.VectorSubcoreMesh(core_axis_name="c", subcore_axis_name="s"),
    scratch_types=[
        pltpu.VMEM((NR, CH), jnp.int32),
        pltpu.VMEM((NR, CH), jnp.int32),
        pltpu.VMEM((NR, CH, D), jnp.float32),
        pltpu.VMEM((TAIL,), jnp.int32),
        pltpu.VMEM((TAIL,), jnp.int32),
        pltpu.VMEM((TAIL, D), jnp.float32),
    ] + [pltpu.SemaphoreType.DMA] * (3 * NR) + [
        pltpu.VMEM_SHARED((N, D), jnp.float32),
    ],
)
def _sc_edge_segsum(z_hbm, src_hbm, dst_hbm, out_hbm,
                    srcb, dstb, rows, srcv_t, dstv_t, rows_t,
                    sg0, sg1, sg2, sg3, si0, si1, si2, si3,
                    ss0, ss1, ss2, ss3, acc):
    _sc_body(z_hbm, src_hbm, dst_hbm, out_hbm,
             srcb, dstb, rows, srcv_t, dstv_t, rows_t,
             (sg0, sg1, sg2, sg3), (si0, si1, si2, si3),
             (ss0, ss1, ss2, ss3), acc)


def kernel(x, edge_index, batch,
           g1_l1_W1, g1_l1_b1, g1_l1_W2, g1_l1_b2,
           g1_l2_W1, g1_l2_b1, g1_l2_W2, g1_l2_b2,
           g2_l1_W1, g2_l1_b1, g2_l1_W2, g2_l1_b2,
           g2_l2_W1, g2_l2_b1, g2_l2_W2, g2_l2_b2,
           ln_W, ln_b):
    f32 = jnp.float32
    H = 64

    src = edge_index[0]
    dst = edge_index[1]

    z64 = jnp.zeros((H, H), f32)
    z1_64 = jnp.zeros((D - 1, H), f32)
    z64_1 = jnp.zeros((1, H), f32)
    wcat = jnp.concatenate([
        jnp.concatenate([z1_64, g2_l1_W1], axis=1),
        jnp.concatenate([g1_l1_W1, z64_1], axis=1),
    ], axis=0)
    bd1 = jnp.concatenate([
        jnp.concatenate([g1_l1_W2, z64], axis=1),
        jnp.concatenate([z64, g2_l1_W2], axis=1),
    ], axis=0)
    bd2 = jnp.concatenate([
        jnp.concatenate([g1_l2_W1, z64], axis=1),
        jnp.concatenate([z64, g2_l2_W1], axis=1),
    ], axis=0)
    bd3 = jnp.concatenate([
        jnp.concatenate([g1_l2_W2, z64], axis=1),
        jnp.concatenate([z64, g2_l2_W2], axis=1),
    ], axis=0)
    b1 = jnp.concatenate([g1_l1_b1, g2_l1_b1]).reshape(1, D)
    b2 = jnp.concatenate([g1_l1_b2, g2_l1_b2]).reshape(1, D)
    b3 = jnp.concatenate([g1_l2_b1, g2_l2_b1]).reshape(1, D)
    b4 = jnp.concatenate([g1_l2_b2, g2_l2_b2]).reshape(1, D)
    lnb = ln_b.reshape(1, G)
    batch3 = batch.reshape(NB, 1, BLK)

    p1 = _sc_edge_segsum(x, src, dst)
    z2 = _tc_mid(x, p1, wcat, bd1, bd2, b1, b2)
    p2 = _sc_edge_segsum(z2, src, dst)
    out = _tc_fin(z2, p2, bd3, b3, b4, ln_W, lnb, batch3)
    return out
